# baseline (speedup 1.0000x reference)
import functools

import jax
import jax.numpy as jnp
from jax import lax
from jax.experimental import pallas as pl
from jax.experimental.pallas import tpu as pltpu
from jax.experimental.pallas import tpu_sc as plsc

N = 10000
E = 320000
D = 128
G = 64
T = 8

NC = 2
NS = 16
NW = NC * NS
L = 128
CPW = 79
CPW_A = 57
CPW_B = 101
EPAD = NW * CPW * L
EROWS = EPAD // L
N_ACC = 10240
RPT_ACC = N_ACC // NS
RPT_OUT = N // NS

B = 1000
GRID = N // B

@functools.cache
def _sc_mesh():
    return plsc.VectorSubcoreMesh(
        core_axis_name="c", subcore_axis_name="s",
        num_cores=NC, num_subcores=NS)


def _stage(dst_ref, src_ref, g):
    for k in range(8):
        dst_ref[0, pl.ds(k * 16, 16)] = src_ref[pl.ds(g * L + k * 16, 16)]


def _sc_hist_body(dst_s, dst_t, out_s, out_t,
                  ones_v, zbuf, idx_v, tab_s, tab_t, sem):
    c = lax.axis_index("c")
    s = lax.axis_index("s")
    w = c * NS + s

    def _fill(i, _):
        ones_v[i, :] = jnp.full((16,), 1.0, jnp.float32)
        zbuf[i, :] = jnp.zeros((16,), jnp.float32)
        return 0
    lax.fori_loop(0, L, _fill, 0)

    def _zero(k, _):
        pltpu.sync_copy(zbuf, tab_s.at[pl.ds(s * RPT_ACC + k * L, L)])
        pltpu.sync_copy(zbuf, tab_t.at[pl.ds(s * RPT_ACC + k * L, L)])
        return 0
    lax.fori_loop(0, RPT_ACC // L, _zero, 0)
    plsc.subcore_barrier()

    def _edges(g, _):
        base = (w * CPW + g) * L
        pltpu.sync_copy(dst_s.at[pl.ds(base, L)], idx_v.at[0])
        pltpu.sync_copy(ones_v, tab_s.at[idx_v.at[0]], add=True)
        pltpu.sync_copy(dst_t.at[pl.ds(base, L)], idx_v.at[0])
        pltpu.sync_copy(ones_v, tab_t.at[idx_v.at[0]], add=True)
        return 0
    lax.fori_loop(0, CPW, _edges, 0)
    plsc.subcore_barrier()

    pltpu.sync_copy(tab_s.at[pl.ds(s * RPT_ACC, RPT_ACC)],
                    out_s.at[c, pl.ds(s * RPT_ACC, RPT_ACC)])
    pltpu.sync_copy(tab_t.at[pl.ds(s * RPT_ACC, RPT_ACC)],
                    out_t.at[c, pl.ds(s * RPT_ACC, RPT_ACC)])


@functools.cache
def _sc_hist():
    return pl.kernel(
        _sc_hist_body,
        out_type=(jax.ShapeDtypeStruct((NC, N_ACC, 16), jnp.float32),
                  jax.ShapeDtypeStruct((NC, N_ACC, 16), jnp.float32)),
        mesh=_sc_mesh(),
        scratch_types=[
            pltpu.VMEM((L, 16), jnp.float32),
            pltpu.VMEM((L, 16), jnp.float32),
            pltpu.VMEM((1, L), jnp.int32),
            pltpu.VMEM_SHARED((N_ACC, 16), jnp.float32),
            pltpu.VMEM_SHARED((N_ACC, 16), jnp.float32),
            pltpu.SemaphoreType.DMA,
        ],
    )


def _sc_spmm_body(hp, src1d, dst1d, out,
                  sb0, sb1, db, r0, r1, acc, g0, g1):
    c = lax.axis_index("c")
    s = lax.axis_index("s")

    def _fill(i, _):
        for k in range(8):
            r0[i, pl.ds(k * 16, 16)] = jnp.zeros((16,), jnp.float32)
        return 0
    lax.fori_loop(0, L, _fill, 0)

    def _zero(k, _):
        pltpu.sync_copy(r0, acc.at[pl.ds(s * RPT_ACC + k * L, L)])
        return 0
    lax.fori_loop(0, RPT_ACC // L, _zero, 0)
    plsc.subcore_barrier()

    def _pipeline(base0, n):
        pltpu.sync_copy(src1d.at[pl.ds(base0, L)], sb0.at[0])
        pltpu.async_copy(hp.at[sb0.at[0]], r0, g0)

        def _edges(k, _):
            j = 2 * k
            pltpu.sync_copy(
                src1d.at[pl.ds(base0 + (j + 1) * L, L)], sb1.at[0])
            pltpu.async_copy(hp.at[sb1.at[0]], r1, g1)
            pltpu.make_async_copy(hp.at[sb0.at[0]], r0, g0).wait()
            pltpu.sync_copy(dst1d.at[pl.ds(base0 + j * L, L)], db.at[0])
            pltpu.sync_copy(r0, acc.at[db.at[0]], add=True)
            pltpu.sync_copy(
                src1d.at[pl.ds(base0 + (j + 2) * L, L)], sb0.at[0])
            pltpu.async_copy(hp.at[sb0.at[0]], r0, g0)
            pltpu.make_async_copy(hp.at[sb1.at[0]], r1, g1).wait()
            pltpu.sync_copy(dst1d.at[pl.ds(base0 + (j + 1) * L, L)], db.at[0])
            pltpu.sync_copy(r1, acc.at[db.at[0]], add=True)
            return 0
        lax.fori_loop(0, (n - 1) // 2, _edges, 0)

        pltpu.make_async_copy(hp.at[sb0.at[0]], r0, g0).wait()
        pltpu.sync_copy(dst1d.at[pl.ds(base0 + (n - 1) * L, L)], db.at[0])
        pltpu.sync_copy(r0, acc.at[db.at[0]], add=True)

    @pl.when(c == 0)
    def _():
        _pipeline((s * CPW_A) * L, CPW_A)

    @pl.when(c == 1)
    def _():
        _pipeline((NS * CPW_A + s * CPW_B) * L, CPW_B)

    plsc.subcore_barrier()

    pltpu.sync_copy(acc.at[pl.ds(s * RPT_ACC, RPT_ACC)],
                    out.at[c, pl.ds(s * RPT_ACC, RPT_ACC)])


@functools.cache
def _sc_spmm():
    return pl.kernel(
        _sc_spmm_body,
        out_type=jax.ShapeDtypeStruct((NC, N_ACC, D), jnp.float32),
        mesh=_sc_mesh(),
        scratch_types=[
            pltpu.VMEM((1, L), jnp.int32),
            pltpu.VMEM((1, L), jnp.int32),
            pltpu.VMEM((1, L), jnp.int32),
            pltpu.VMEM((L, D), jnp.float32),
            pltpu.VMEM((L, D), jnp.float32),
            pltpu.VMEM_SHARED((N_ACC, D), jnp.float32),
            pltpu.SemaphoreType.DMA,
            pltpu.SemaphoreType.DMA,
        ],
    )


def _dinv(degp_ref):
    deg = degp_ref[0, :, 0:1] + degp_ref[1, :, 0:1] + 1.0
    return lax.rsqrt(deg)


def _pre_body(x_ref, w_ref, degp_ref, o_ref):
    o_ref[...] = _dinv(degp_ref) * jnp.dot(
        x_ref[...], w_ref[...], preferred_element_type=jnp.float32)


@functools.cache
def _tc_pre():
    return pl.pallas_call(
    _pre_body,
    grid=(GRID,),
    in_specs=[
        pl.BlockSpec((B, D), lambda i: (i, 0)),
        pl.BlockSpec((D, D), lambda i: (0, 0)),
        pl.BlockSpec((NC, B, 16), lambda i: (0, i, 0)),
    ],
    out_specs=pl.BlockSpec((B, D), lambda i: (i, 0)),
    out_shape=jax.ShapeDtypeStruct((N, D), jnp.float32),
)


def _mid_body(accp_ref, hp1_ref, degp_ref, b1_ref, w2_ref, o_ref):
    dinv = _dinv(degp_ref)
    h1 = dinv * (accp_ref[0] + accp_ref[1] + hp1_ref[...]) + b1_ref[...]
    h1 = jnp.maximum(h1, 0.0)
    o_ref[...] = dinv * jnp.dot(h1, w2_ref[...],
                                preferred_element_type=jnp.float32)


@functools.cache
def _tc_mid():
    return pl.pallas_call(
    _mid_body,
    grid=(GRID,),
    in_specs=[
        pl.BlockSpec((NC, B, D), lambda i: (0, i, 0)),
        pl.BlockSpec((B, D), lambda i: (i, 0)),
        pl.BlockSpec((NC, B, 16), lambda i: (0, i, 0)),
        pl.BlockSpec((1, D), lambda i: (0, 0)),
        pl.BlockSpec((D, D), lambda i: (0, 0)),
    ],
    out_specs=pl.BlockSpec((B, D), lambda i: (i, 0)),
    out_shape=jax.ShapeDtypeStruct((N, D), jnp.float32),
)


def _post_s_body(accp_ref, hp2_ref, degp_ref, b2_ref, xs_ref, o_ref):
    dinv = _dinv(degp_ref)
    o_ref[...] = (dinv * (accp_ref[0] + accp_ref[1] + hp2_ref[...])
                  + b2_ref[...] + xs_ref[...])


@functools.cache
def _tc_post_s():
    return pl.pallas_call(
    _post_s_body,
    grid=(GRID,),
    in_specs=[
        pl.BlockSpec((NC, B, D), lambda i: (0, i, 0)),
        pl.BlockSpec((B, D), lambda i: (i, 0)),
        pl.BlockSpec((NC, B, 16), lambda i: (0, i, 0)),
        pl.BlockSpec((1, D), lambda i: (0, 0)),
        pl.BlockSpec((B, D), lambda i: (i, 0)),
    ],
    out_specs=pl.BlockSpec((B, D), lambda i: (i, 0)),
    out_shape=jax.ShapeDtypeStruct((N, D), jnp.float32),
)


def _post_t_body(accp_ref, hp2_ref, degp_ref, b2_ref, tok_ref, o_ref):
    dinv = _dinv(degp_ref)
    toksum = jnp.sum(tok_ref[...], axis=1)
    o_ref[...] = (dinv * (accp_ref[0] + accp_ref[1] + hp2_ref[...])
                  + b2_ref[...] + toksum * 0.125)


@functools.cache
def _tc_post_t():
    return pl.pallas_call(
    _post_t_body,
    grid=(GRID,),
    in_specs=[
        pl.BlockSpec((NC, B, D), lambda i: (0, i, 0)),
        pl.BlockSpec((B, D), lambda i: (i, 0)),
        pl.BlockSpec((NC, B, 16), lambda i: (0, i, 0)),
        pl.BlockSpec((1, D), lambda i: (0, 0)),
        pl.BlockSpec((B, T, D), lambda i: (i, 0, 0)),
    ],
    out_specs=pl.BlockSpec((B, D), lambda i: (i, 0)),
    out_shape=jax.ShapeDtypeStruct((N, D), jnp.float32),
)


def _pool_body(hs_ref, m_ref, b_ref, ss_ref, sm_ref, cnt_ref):
    i = pl.program_id(0)
    oh = (b_ref[...] == lax.broadcasted_iota(jnp.int32, (B, G), 1)
          ).astype(jnp.float32)
    dn = (((0,), (0,)), ((), ()))
    ss = lax.dot_general(oh, hs_ref[...], dn,
                         preferred_element_type=jnp.float32)
    sm = lax.dot_general(oh, m_ref[...], dn,
                         preferred_element_type=jnp.float32)
    cn = jnp.broadcast_to(jnp.sum(oh, axis=0)[:, None], (G, D))

    @pl.when(i == 0)
    def _():
        ss_ref[...] = ss
        sm_ref[...] = sm
        cnt_ref[...] = cn

    @pl.when(i > 0)
    def _():
        ss_ref[...] += ss
        sm_ref[...] += sm
        cnt_ref[...] += cn


@functools.cache
def _tc_pool():
    return pl.pallas_call(
    _pool_body,
    grid=(GRID,),
    in_specs=[
        pl.BlockSpec((B, D), lambda i: (i, 0)),
        pl.BlockSpec((B, D), lambda i: (i, 0)),
        pl.BlockSpec((B, 1), lambda i: (i, 0)),
    ],
    out_specs=[
        pl.BlockSpec((G, D), lambda i: (0, 0)),
        pl.BlockSpec((G, D), lambda i: (0, 0)),
        pl.BlockSpec((G, D), lambda i: (0, 0)),
    ],
    out_shape=[jax.ShapeDtypeStruct((G, D), jnp.float32)] * 3,
)

_TFACT = 8.0 / (8.0 + 1e-8)


def _scores_body(hs_ref, m_ref, bp_ref, bn_ref, ss_ref, sm_ref, cnt_ref,
                 osp_ref, otp_ref, osn_ref, otn_ref, stat_ref):
    i = pl.program_id(0)
    inv_cnt = 1.0 / jnp.maximum(cnt_ref[...], 1.0)
    p_s = ss_ref[...] * inv_cnt
    p_t = sm_ref[...] * inv_cnt
    dn = (((1,), (1,)), ((), ()))
    z_s = lax.dot_general(hs_ref[...], p_s, dn,
                          preferred_element_type=jnp.float32)
    z_t = lax.dot_general(m_ref[...], p_t, dn,
                          preferred_element_type=jnp.float32) * _TFACT
    iot = lax.broadcasted_iota(jnp.int32, (B, G), 1)
    ohp = bp_ref[...] == iot
    ohn = bn_ref[...] == iot
    sp = jnp.sum(jnp.where(ohp, z_s, 0.0), axis=1, keepdims=True)
    tp = jnp.sum(jnp.where(ohp, z_t, 0.0), axis=1, keepdims=True)
    sn = jnp.sum(jnp.where(ohn, z_s, 0.0), axis=1, keepdims=True)
    tn = jnp.sum(jnp.where(ohn, z_t, 0.0), axis=1, keepdims=True)
    osp_ref[...] = sp
    otp_ref[...] = tp
    osn_ref[...] = sn
    otn_ref[...] = tn

    vals = (jnp.sum(sp), jnp.sum(tp), jnp.sum(sp * sp), jnp.sum(tp * tp),
            jnp.sum(sp * tp), jnp.sum(sn), jnp.sum(tn), jnp.sum(sn * sn),
            jnp.sum(tn * tn), jnp.sum(sn * tn))

    @pl.when(i == 0)
    def _():
        for k in range(10):
            stat_ref[k] = vals[k]

    @pl.when(i > 0)
    def _():
        for k in range(10):
            stat_ref[k] += vals[k]


@functools.cache
def _tc_scores():
    return pl.pallas_call(
    _scores_body,
    grid=(GRID,),
    in_specs=[
        pl.BlockSpec((B, D), lambda i: (i, 0)),
        pl.BlockSpec((B, D), lambda i: (i, 0)),
        pl.BlockSpec((B, 1), lambda i: (i, 0)),
        pl.BlockSpec((B, 1), lambda i: (i, 0)),
        pl.BlockSpec((G, D), lambda i: (0, 0)),
        pl.BlockSpec((G, D), lambda i: (0, 0)),
        pl.BlockSpec((G, D), lambda i: (0, 0)),
    ],
    out_specs=[
        pl.BlockSpec((B, 1), lambda i: (i, 0)),
        pl.BlockSpec((B, 1), lambda i: (i, 0)),
        pl.BlockSpec((B, 1), lambda i: (i, 0)),
        pl.BlockSpec((B, 1), lambda i: (i, 0)),
        pl.BlockSpec(memory_space=pltpu.SMEM),
    ],
    out_shape=[jax.ShapeDtypeStruct((N, 1), jnp.float32)] * 4
    + [jax.ShapeDtypeStruct((16,), jnp.float32)],
)


def _fuse_body(sp_ref, tp_ref, sn_ref, tn_ref, stat_ref, op_ref, on_ref):
    n = jnp.float32(N)

    def _std(s1, s2):
        mu = s1 / n
        var = (s2 - n * mu * mu) / (n - 1.0)
        return mu, jnp.sqrt(var) + 1e-10

    st = stat_ref
    mu_sp, sd_sp = _std(st[0], st[2])
    mu_tp, sd_tp = _std(st[1], st[3])
    cov_p = (st[4] - n * mu_sp * mu_tp) / (n * sd_sp * sd_tp)
    mu_sn, sd_sn = _std(st[5], st[7])
    mu_tn, sd_tn = _std(st[6], st[8])
    cov_n = (st[9] - n * mu_sn * mu_tn) / (n * sd_sn * sd_tn)

    ap = (sp_ref[...] - mu_sp) / sd_sp
    bp = (tp_ref[...] - mu_tp) / sd_tp
    op_ref[...] = jax.nn.sigmoid(ap + cov_p * bp)
    an = (sn_ref[...] - mu_sn) / sd_sn
    bn = (tn_ref[...] - mu_tn) / sd_tn
    on_ref[...] = jax.nn.sigmoid(an + cov_n * bn)


@functools.cache
def _tc_fuse():
    return pl.pallas_call(
    _fuse_body,
    grid=(GRID,),
    in_specs=[
        pl.BlockSpec((B, 1), lambda i: (i, 0)),
        pl.BlockSpec((B, 1), lambda i: (i, 0)),
        pl.BlockSpec((B, 1), lambda i: (i, 0)),
        pl.BlockSpec((B, 1), lambda i: (i, 0)),
        pl.BlockSpec(memory_space=pltpu.SMEM),
    ],
    out_specs=[
        pl.BlockSpec((B, 1), lambda i: (i, 0)),
        pl.BlockSpec((B, 1), lambda i: (i, 0)),
    ],
    out_shape=[jax.ShapeDtypeStruct((N, 1), jnp.float32)] * 2,
)


def _pad_edges(ei):
    src = jnp.concatenate(
        [ei[0].astype(jnp.int32), jnp.zeros((EPAD - E,), jnp.int32)])
    dst = jnp.concatenate(
        [ei[1].astype(jnp.int32), jnp.full((EPAD - E,), N, jnp.int32)])
    return src, dst


def kernel(x_s, edge_index_s, x_t, edge_index_t, tokens, batch, neg_batch,
           W_s1, b_s1, W_s2, b_s2, W_t1, b_t1, W_t2, b_t2):
    src_s, dst_s = _pad_edges(edge_index_s)
    src_t, dst_t = _pad_edges(edge_index_t)

    degp_s, degp_t = _sc_hist()(dst_s, dst_t)

    hp_s1 = _tc_pre()(x_s, W_s1, degp_s)
    hp_t1 = _tc_pre()(x_t, W_t1, degp_t)

    acc_s1 = _sc_spmm()(hp_s1, src_s, dst_s)
    acc_t1 = _sc_spmm()(hp_t1, src_t, dst_t)

    hp_s2 = _tc_mid()(acc_s1, hp_s1, degp_s, b_s1.reshape(1, D), W_s2)
    hp_t2 = _tc_mid()(acc_t1, hp_t1, degp_t, b_t1.reshape(1, D), W_t2)

    acc_s2 = _sc_spmm()(hp_s2, src_s, dst_s)
    acc_t2 = _sc_spmm()(hp_t2, src_t, dst_t)

    h_s = _tc_post_s()(acc_s2, hp_s2, degp_s, b_s2.reshape(1, D), x_s)
    m = _tc_post_t()(acc_t2, hp_t2, degp_t, b_t2.reshape(1, D), tokens)

    batch2 = batch.astype(jnp.int32).reshape(N, 1)
    neg2 = neg_batch.astype(jnp.int32).reshape(N, 1)

    sums_s, sums_m, cnts = _tc_pool()(h_s, m, batch2)
    sp, tp, sn, tn, stats = _tc_scores()(h_s, m, batch2, neg2,
                                       sums_s, sums_m, cnts)
    s_pos, s_neg = _tc_fuse()(sp, tp, sn, tn, stats)
    return (s_pos, s_neg)


# speedup vs baseline: 1.1312x; 1.1312x over previous
import functools

import jax
import jax.numpy as jnp
from jax import lax
from jax.experimental import pallas as pl
from jax.experimental.pallas import tpu as pltpu
from jax.experimental.pallas import tpu_sc as plsc

N = 10000
E = 320000
D = 128
G = 64
T = 8

NC = 2
NS = 16
NW = NC * NS
L = 128
CPW = 79
CPW_A = 101
CPW_B = 57
EPAD = NW * CPW * L
EROWS = EPAD // L
N_ACC = 10240
RPT_ACC = N_ACC // NS
RPT_OUT = N // NS

B = 1000
GRID = N // B

@functools.cache
def _sc_mesh():
    return plsc.VectorSubcoreMesh(
        core_axis_name="c", subcore_axis_name="s",
        num_cores=NC, num_subcores=NS)


def _stage(dst_ref, src_ref, g):
    for k in range(8):
        dst_ref[0, pl.ds(k * 16, 16)] = src_ref[pl.ds(g * L + k * 16, 16)]


def _sc_hist_body(dst_s, dst_t, out_s, out_t,
                  ones_v, zbuf, idx_v, tab_s, tab_t, sem):
    c = lax.axis_index("c")
    s = lax.axis_index("s")
    w = c * NS + s

    def _fill(i, _):
        ones_v[i, :] = jnp.full((16,), 1.0, jnp.float32)
        zbuf[i, :] = jnp.zeros((16,), jnp.float32)
        return 0
    lax.fori_loop(0, L, _fill, 0)

    def _zero(k, _):
        pltpu.sync_copy(zbuf, tab_s.at[pl.ds(s * RPT_ACC + k * L, L)])
        pltpu.sync_copy(zbuf, tab_t.at[pl.ds(s * RPT_ACC + k * L, L)])
        return 0
    lax.fori_loop(0, RPT_ACC // L, _zero, 0)
    plsc.subcore_barrier()

    def _edges(g, _):
        base = (w * CPW + g) * L
        pltpu.sync_copy(dst_s.at[pl.ds(base, L)], idx_v.at[0])
        pltpu.sync_copy(ones_v, tab_s.at[idx_v.at[0]], add=True)
        pltpu.sync_copy(dst_t.at[pl.ds(base, L)], idx_v.at[0])
        pltpu.sync_copy(ones_v, tab_t.at[idx_v.at[0]], add=True)
        return 0
    lax.fori_loop(0, CPW, _edges, 0)
    plsc.subcore_barrier()

    pltpu.sync_copy(tab_s.at[pl.ds(s * RPT_ACC, RPT_ACC)],
                    out_s.at[c, pl.ds(s * RPT_ACC, RPT_ACC)])
    pltpu.sync_copy(tab_t.at[pl.ds(s * RPT_ACC, RPT_ACC)],
                    out_t.at[c, pl.ds(s * RPT_ACC, RPT_ACC)])


@functools.cache
def _sc_hist():
    return pl.kernel(
        _sc_hist_body,
        out_type=(jax.ShapeDtypeStruct((NC, N_ACC, 16), jnp.float32),
                  jax.ShapeDtypeStruct((NC, N_ACC, 16), jnp.float32)),
        mesh=_sc_mesh(),
        scratch_types=[
            pltpu.VMEM((L, 16), jnp.float32),
            pltpu.VMEM((L, 16), jnp.float32),
            pltpu.VMEM((1, L), jnp.int32),
            pltpu.VMEM_SHARED((N_ACC, 16), jnp.float32),
            pltpu.VMEM_SHARED((N_ACC, 16), jnp.float32),
            pltpu.SemaphoreType.DMA,
        ],
    )


def _sc_spmm_body(hp, src1d, dst1d, out,
                  sb0, sb1, db, r0, r1, acc, g0, g1):
    c = lax.axis_index("c")
    s = lax.axis_index("s")

    def _fill(i, _):
        for k in range(8):
            r0[i, pl.ds(k * 16, 16)] = jnp.zeros((16,), jnp.float32)
        return 0
    lax.fori_loop(0, L, _fill, 0)

    def _zero(k, _):
        pltpu.sync_copy(r0, acc.at[pl.ds(s * RPT_ACC + k * L, L)])
        return 0
    lax.fori_loop(0, RPT_ACC // L, _zero, 0)
    plsc.subcore_barrier()

    def _pipeline(base0, n):
        pltpu.sync_copy(src1d.at[pl.ds(base0, L)], sb0.at[0])
        pltpu.async_copy(hp.at[sb0.at[0]], r0, g0)

        def _edges(k, _):
            j = 2 * k
            pltpu.sync_copy(
                src1d.at[pl.ds(base0 + (j + 1) * L, L)], sb1.at[0])
            pltpu.async_copy(hp.at[sb1.at[0]], r1, g1)
            pltpu.make_async_copy(hp.at[sb0.at[0]], r0, g0).wait()
            pltpu.sync_copy(dst1d.at[pl.ds(base0 + j * L, L)], db.at[0])
            pltpu.sync_copy(r0, acc.at[db.at[0]], add=True)
            pltpu.sync_copy(
                src1d.at[pl.ds(base0 + (j + 2) * L, L)], sb0.at[0])
            pltpu.async_copy(hp.at[sb0.at[0]], r0, g0)
            pltpu.make_async_copy(hp.at[sb1.at[0]], r1, g1).wait()
            pltpu.sync_copy(dst1d.at[pl.ds(base0 + (j + 1) * L, L)], db.at[0])
            pltpu.sync_copy(r1, acc.at[db.at[0]], add=True)
            return 0
        lax.fori_loop(0, (n - 1) // 2, _edges, 0)

        pltpu.make_async_copy(hp.at[sb0.at[0]], r0, g0).wait()
        pltpu.sync_copy(dst1d.at[pl.ds(base0 + (n - 1) * L, L)], db.at[0])
        pltpu.sync_copy(r0, acc.at[db.at[0]], add=True)

    @pl.when(c == 0)
    def _():
        _pipeline((s * CPW_A) * L, CPW_A)

    @pl.when(c == 1)
    def _():
        _pipeline((NS * CPW_A + s * CPW_B) * L, CPW_B)

    plsc.subcore_barrier()

    pltpu.sync_copy(acc.at[pl.ds(s * RPT_ACC, RPT_ACC)],
                    out.at[c, pl.ds(s * RPT_ACC, RPT_ACC)])


@functools.cache
def _sc_spmm():
    return pl.kernel(
        _sc_spmm_body,
        out_type=jax.ShapeDtypeStruct((NC, N_ACC, D), jnp.float32),
        mesh=_sc_mesh(),
        scratch_types=[
            pltpu.VMEM((1, L), jnp.int32),
            pltpu.VMEM((1, L), jnp.int32),
            pltpu.VMEM((1, L), jnp.int32),
            pltpu.VMEM((L, D), jnp.float32),
            pltpu.VMEM((L, D), jnp.float32),
            pltpu.VMEM_SHARED((N_ACC, D), jnp.float32),
            pltpu.SemaphoreType.DMA,
            pltpu.SemaphoreType.DMA,
        ],
    )


def _dinv(degp_ref):
    deg = degp_ref[0, :, 0:1] + degp_ref[1, :, 0:1] + 1.0
    return lax.rsqrt(deg)


def _pre_body(x_ref, w_ref, degp_ref, o_ref):
    o_ref[...] = _dinv(degp_ref) * jnp.dot(
        x_ref[...], w_ref[...], preferred_element_type=jnp.float32)


@functools.cache
def _tc_pre():
    return pl.pallas_call(
    _pre_body,
    grid=(GRID,),
    in_specs=[
        pl.BlockSpec((B, D), lambda i: (i, 0)),
        pl.BlockSpec((D, D), lambda i: (0, 0)),
        pl.BlockSpec((NC, B, 16), lambda i: (0, i, 0)),
    ],
    out_specs=pl.BlockSpec((B, D), lambda i: (i, 0)),
    out_shape=jax.ShapeDtypeStruct((N, D), jnp.float32),
)


def _mid_body(accp_ref, hp1_ref, degp_ref, b1_ref, w2_ref, o_ref):
    dinv = _dinv(degp_ref)
    h1 = dinv * (accp_ref[0] + accp_ref[1] + hp1_ref[...]) + b1_ref[...]
    h1 = jnp.maximum(h1, 0.0)
    o_ref[...] = dinv * jnp.dot(h1, w2_ref[...],
                                preferred_element_type=jnp.float32)


@functools.cache
def _tc_mid():
    return pl.pallas_call(
    _mid_body,
    grid=(GRID,),
    in_specs=[
        pl.BlockSpec((NC, B, D), lambda i: (0, i, 0)),
        pl.BlockSpec((B, D), lambda i: (i, 0)),
        pl.BlockSpec((NC, B, 16), lambda i: (0, i, 0)),
        pl.BlockSpec((1, D), lambda i: (0, 0)),
        pl.BlockSpec((D, D), lambda i: (0, 0)),
    ],
    out_specs=pl.BlockSpec((B, D), lambda i: (i, 0)),
    out_shape=jax.ShapeDtypeStruct((N, D), jnp.float32),
)


def _post_s_body(accp_ref, hp2_ref, degp_ref, b2_ref, xs_ref, o_ref):
    dinv = _dinv(degp_ref)
    o_ref[...] = (dinv * (accp_ref[0] + accp_ref[1] + hp2_ref[...])
                  + b2_ref[...] + xs_ref[...])


@functools.cache
def _tc_post_s():
    return pl.pallas_call(
    _post_s_body,
    grid=(GRID,),
    in_specs=[
        pl.BlockSpec((NC, B, D), lambda i: (0, i, 0)),
        pl.BlockSpec((B, D), lambda i: (i, 0)),
        pl.BlockSpec((NC, B, 16), lambda i: (0, i, 0)),
        pl.BlockSpec((1, D), lambda i: (0, 0)),
        pl.BlockSpec((B, D), lambda i: (i, 0)),
    ],
    out_specs=pl.BlockSpec((B, D), lambda i: (i, 0)),
    out_shape=jax.ShapeDtypeStruct((N, D), jnp.float32),
)


def _post_t_body(accp_ref, hp2_ref, degp_ref, b2_ref, tok_ref, o_ref):
    dinv = _dinv(degp_ref)
    toksum = jnp.sum(tok_ref[...], axis=1)
    o_ref[...] = (dinv * (accp_ref[0] + accp_ref[1] + hp2_ref[...])
                  + b2_ref[...] + toksum * 0.125)


@functools.cache
def _tc_post_t():
    return pl.pallas_call(
    _post_t_body,
    grid=(GRID,),
    in_specs=[
        pl.BlockSpec((NC, B, D), lambda i: (0, i, 0)),
        pl.BlockSpec((B, D), lambda i: (i, 0)),
        pl.BlockSpec((NC, B, 16), lambda i: (0, i, 0)),
        pl.BlockSpec((1, D), lambda i: (0, 0)),
        pl.BlockSpec((B, T, D), lambda i: (i, 0, 0)),
    ],
    out_specs=pl.BlockSpec((B, D), lambda i: (i, 0)),
    out_shape=jax.ShapeDtypeStruct((N, D), jnp.float32),
)


def _pool_body(hs_ref, m_ref, b_ref, ss_ref, sm_ref, cnt_ref):
    i = pl.program_id(0)
    oh = (b_ref[...] == lax.broadcasted_iota(jnp.int32, (B, G), 1)
          ).astype(jnp.float32)
    dn = (((0,), (0,)), ((), ()))
    ss = lax.dot_general(oh, hs_ref[...], dn,
                         preferred_element_type=jnp.float32)
    sm = lax.dot_general(oh, m_ref[...], dn,
                         preferred_element_type=jnp.float32)
    cn = jnp.broadcast_to(jnp.sum(oh, axis=0)[:, None], (G, D))

    @pl.when(i == 0)
    def _():
        ss_ref[...] = ss
        sm_ref[...] = sm
        cnt_ref[...] = cn

    @pl.when(i > 0)
    def _():
        ss_ref[...] += ss
        sm_ref[...] += sm
        cnt_ref[...] += cn


@functools.cache
def _tc_pool():
    return pl.pallas_call(
    _pool_body,
    grid=(GRID,),
    in_specs=[
        pl.BlockSpec((B, D), lambda i: (i, 0)),
        pl.BlockSpec((B, D), lambda i: (i, 0)),
        pl.BlockSpec((B, 1), lambda i: (i, 0)),
    ],
    out_specs=[
        pl.BlockSpec((G, D), lambda i: (0, 0)),
        pl.BlockSpec((G, D), lambda i: (0, 0)),
        pl.BlockSpec((G, D), lambda i: (0, 0)),
    ],
    out_shape=[jax.ShapeDtypeStruct((G, D), jnp.float32)] * 3,
)

_TFACT = 8.0 / (8.0 + 1e-8)


def _scores_body(hs_ref, m_ref, bp_ref, bn_ref, ss_ref, sm_ref, cnt_ref,
                 osp_ref, otp_ref, osn_ref, otn_ref, stat_ref):
    i = pl.program_id(0)
    inv_cnt = 1.0 / jnp.maximum(cnt_ref[...], 1.0)
    p_s = ss_ref[...] * inv_cnt
    p_t = sm_ref[...] * inv_cnt
    dn = (((1,), (1,)), ((), ()))
    z_s = lax.dot_general(hs_ref[...], p_s, dn,
                          preferred_element_type=jnp.float32)
    z_t = lax.dot_general(m_ref[...], p_t, dn,
                          preferred_element_type=jnp.float32) * _TFACT
    iot = lax.broadcasted_iota(jnp.int32, (B, G), 1)
    ohp = bp_ref[...] == iot
    ohn = bn_ref[...] == iot
    sp = jnp.sum(jnp.where(ohp, z_s, 0.0), axis=1, keepdims=True)
    tp = jnp.sum(jnp.where(ohp, z_t, 0.0), axis=1, keepdims=True)
    sn = jnp.sum(jnp.where(ohn, z_s, 0.0), axis=1, keepdims=True)
    tn = jnp.sum(jnp.where(ohn, z_t, 0.0), axis=1, keepdims=True)
    osp_ref[...] = sp
    otp_ref[...] = tp
    osn_ref[...] = sn
    otn_ref[...] = tn

    vals = (jnp.sum(sp), jnp.sum(tp), jnp.sum(sp * sp), jnp.sum(tp * tp),
            jnp.sum(sp * tp), jnp.sum(sn), jnp.sum(tn), jnp.sum(sn * sn),
            jnp.sum(tn * tn), jnp.sum(sn * tn))

    @pl.when(i == 0)
    def _():
        for k in range(10):
            stat_ref[k] = vals[k]

    @pl.when(i > 0)
    def _():
        for k in range(10):
            stat_ref[k] += vals[k]


@functools.cache
def _tc_scores():
    return pl.pallas_call(
    _scores_body,
    grid=(GRID,),
    in_specs=[
        pl.BlockSpec((B, D), lambda i: (i, 0)),
        pl.BlockSpec((B, D), lambda i: (i, 0)),
        pl.BlockSpec((B, 1), lambda i: (i, 0)),
        pl.BlockSpec((B, 1), lambda i: (i, 0)),
        pl.BlockSpec((G, D), lambda i: (0, 0)),
        pl.BlockSpec((G, D), lambda i: (0, 0)),
        pl.BlockSpec((G, D), lambda i: (0, 0)),
    ],
    out_specs=[
        pl.BlockSpec((B, 1), lambda i: (i, 0)),
        pl.BlockSpec((B, 1), lambda i: (i, 0)),
        pl.BlockSpec((B, 1), lambda i: (i, 0)),
        pl.BlockSpec((B, 1), lambda i: (i, 0)),
        pl.BlockSpec(memory_space=pltpu.SMEM),
    ],
    out_shape=[jax.ShapeDtypeStruct((N, 1), jnp.float32)] * 4
    + [jax.ShapeDtypeStruct((16,), jnp.float32)],
)


def _fuse_body(sp_ref, tp_ref, sn_ref, tn_ref, stat_ref, op_ref, on_ref):
    n = jnp.float32(N)

    def _std(s1, s2):
        mu = s1 / n
        var = (s2 - n * mu * mu) / (n - 1.0)
        return mu, jnp.sqrt(var) + 1e-10

    st = stat_ref
    mu_sp, sd_sp = _std(st[0], st[2])
    mu_tp, sd_tp = _std(st[1], st[3])
    cov_p = (st[4] - n * mu_sp * mu_tp) / (n * sd_sp * sd_tp)
    mu_sn, sd_sn = _std(st[5], st[7])
    mu_tn, sd_tn = _std(st[6], st[8])
    cov_n = (st[9] - n * mu_sn * mu_tn) / (n * sd_sn * sd_tn)

    ap = (sp_ref[...] - mu_sp) / sd_sp
    bp = (tp_ref[...] - mu_tp) / sd_tp
    op_ref[...] = jax.nn.sigmoid(ap + cov_p * bp)
    an = (sn_ref[...] - mu_sn) / sd_sn
    bn = (tn_ref[...] - mu_tn) / sd_tn
    on_ref[...] = jax.nn.sigmoid(an + cov_n * bn)


@functools.cache
def _tc_fuse():
    return pl.pallas_call(
    _fuse_body,
    grid=(GRID,),
    in_specs=[
        pl.BlockSpec((B, 1), lambda i: (i, 0)),
        pl.BlockSpec((B, 1), lambda i: (i, 0)),
        pl.BlockSpec((B, 1), lambda i: (i, 0)),
        pl.BlockSpec((B, 1), lambda i: (i, 0)),
        pl.BlockSpec(memory_space=pltpu.SMEM),
    ],
    out_specs=[
        pl.BlockSpec((B, 1), lambda i: (i, 0)),
        pl.BlockSpec((B, 1), lambda i: (i, 0)),
    ],
    out_shape=[jax.ShapeDtypeStruct((N, 1), jnp.float32)] * 2,
)


def _pad_edges(ei):
    src = jnp.concatenate(
        [ei[0].astype(jnp.int32), jnp.zeros((EPAD - E,), jnp.int32)])
    dst = jnp.concatenate(
        [ei[1].astype(jnp.int32), jnp.full((EPAD - E,), N, jnp.int32)])
    return src, dst


def kernel(x_s, edge_index_s, x_t, edge_index_t, tokens, batch, neg_batch,
           W_s1, b_s1, W_s2, b_s2, W_t1, b_t1, W_t2, b_t2):
    src_s, dst_s = _pad_edges(edge_index_s)
    src_t, dst_t = _pad_edges(edge_index_t)

    degp_s, degp_t = _sc_hist()(dst_s, dst_t)

    hp_s1 = _tc_pre()(x_s, W_s1, degp_s)
    hp_t1 = _tc_pre()(x_t, W_t1, degp_t)

    acc_s1 = _sc_spmm()(hp_s1, src_s, dst_s)
    acc_t1 = _sc_spmm()(hp_t1, src_t, dst_t)

    hp_s2 = _tc_mid()(acc_s1, hp_s1, degp_s, b_s1.reshape(1, D), W_s2)
    hp_t2 = _tc_mid()(acc_t1, hp_t1, degp_t, b_t1.reshape(1, D), W_t2)

    acc_s2 = _sc_spmm()(hp_s2, src_s, dst_s)
    acc_t2 = _sc_spmm()(hp_t2, src_t, dst_t)

    h_s = _tc_post_s()(acc_s2, hp_s2, degp_s, b_s2.reshape(1, D), x_s)
    m = _tc_post_t()(acc_t2, hp_t2, degp_t, b_t2.reshape(1, D), tokens)

    batch2 = batch.astype(jnp.int32).reshape(N, 1)
    neg2 = neg_batch.astype(jnp.int32).reshape(N, 1)

    sums_s, sums_m, cnts = _tc_pool()(h_s, m, batch2)
    sp, tp, sn, tn, stats = _tc_scores()(h_s, m, batch2, neg2,
                                       sums_s, sums_m, cnts)
    s_pos, s_neg = _tc_fuse()(sp, tp, sn, tn, stats)
    return (s_pos, s_neg)
